# baseline (speedup 1.0000x reference)
import functools

import jax
import jax.numpy as jnp
from jax import lax
from jax.experimental import pallas as pl
from jax.experimental.pallas import tpu as pltpu
from jax.experimental.pallas import tpu_sc as plsc

N = 10000
E = 320000
D = 128
G = 64
BN_EPS = 1e-5

NC = 2
NS = 16
DH = D // NC
CL = 128
NCH = 160
EPW = CL * NCH
EPAD = EPW * NS
NPAD = 10112
RPT = NPAD // NS
NLAST = N - (NS - 1) * RPT

KBUF = 4
MG = 1
NQ = 4
NCHQ = NCH // NQ

BR = 400
NBLK = N // BR
INV_BN = 1.0 / (1.0 + BN_EPS) ** 0.5


def _sc_aggregate(xn, src_r, dst_r, zeros_hbm):
    mesh = plsc.VectorSubcoreMesh(core_axis_name="c", subcore_axis_name="s")

    @functools.partial(
        pl.kernel,
        mesh=mesh,
        compiler_params=pltpu.CompilerParams(use_tc_tiling_on_sc=False),
        out_type=jax.ShapeDtypeStruct((NC, NPAD, DH), jnp.float32),
        scratch_types=[
            pltpu.VMEM((NCHQ, CL), jnp.int32),
            pltpu.VMEM((NCHQ, CL), jnp.int32),
            [pltpu.VMEM((CL, DH), jnp.float32) for _ in range(KBUF)],
            pltpu.VMEM_SHARED((NPAD, DH), jnp.float32),
            pltpu.VMEM_SHARED((NPAD, DH), jnp.float32),
            [pltpu.SemaphoreType.DMA for _ in range(KBUF)],
            [pltpu.SemaphoreType.DMA for _ in range(KBUF)],
        ],
    )
    def agg_kernel(x_hbm, src_hbm, dst_hbm, z_hbm, out_hbm,
                   src_v, dst_v, rows, tab_sh, agg_sh, gsem, ssem):
        cid = lax.axis_index("c")
        sid = lax.axis_index("s")
        rs = pl.ds(sid * RPT, RPT)
        cs = pl.ds(cid * DH, DH)
        @pl.when(sid < NS - 1)
        def _():
            pltpu.sync_copy(x_hbm.at[pl.ds(sid * RPT, RPT), cs],
                            tab_sh.at[rs])

        @pl.when(sid == NS - 1)
        def _():
            pltpu.sync_copy(x_hbm.at[pl.ds((NS - 1) * RPT, NLAST), cs],
                            tab_sh.at[pl.ds((NS - 1) * RPT, NLAST)])

        pltpu.sync_copy(z_hbm.at[rs], agg_sh.at[rs])
        plsc.subcore_barrier()

        for q in range(NQ):
            pltpu.sync_copy(src_hbm.at[sid, pl.ds(q * NCHQ, NCHQ)], src_v)
            pltpu.sync_copy(dst_hbm.at[sid, pl.ds(q * NCHQ, NCHQ)], dst_v)
            for b in range(KBUF):
                pltpu.async_copy(tab_sh.at[src_v.at[b]], rows[b], gsem[b])

            def outer(j0, carry):
                for b in range(KBUF):
                    j = j0 * KBUF + b
                    pltpu.make_async_copy(tab_sh.at[src_v.at[0]],
                                          rows[b], gsem[b]).wait()
                    pltpu.async_copy(rows[b], agg_sh.at[dst_v.at[j]],
                                     ssem[b], add=True)
                    jf = j + MG
                    bf = (b + MG) % KBUF

                    @pl.when(jnp.logical_and(jf >= KBUF, jf < NCHQ))
                    def _():
                        pltpu.make_async_copy(rows[bf],
                                              agg_sh.at[dst_v.at[0]],
                                              ssem[bf]).wait()
                        pltpu.async_copy(tab_sh.at[src_v.at[jf]],
                                         rows[bf], gsem[bf])
                return carry

            lax.fori_loop(0, NCHQ // KBUF, outer, 0)
            for b in range(KBUF):
                pltpu.make_async_copy(rows[b], agg_sh.at[dst_v.at[0]],
                                      ssem[b]).wait()
        plsc.subcore_barrier()
        pltpu.sync_copy(agg_sh.at[rs], out_hbm.at[cid, rs])

    return agg_kernel(xn, src_r, dst_r, zeros_hbm)


def _mlp_res_block(x, agg, Wa, ba, Wb, bb, scale, be):
    def body(x_ref, a_ref, Wa_ref, ba_ref, Wb_ref, bb_ref,
             s_ref, be_ref, o_ref):
        xb = x_ref[...]
        h = xb + jnp.concatenate([a_ref[0], a_ref[1]], axis=1)
        t = jnp.dot(h, Wa_ref[...], preferred_element_type=jnp.float32)
        t = jnp.maximum(t + ba_ref[...], 0.0)
        u = jnp.dot(t, Wb_ref[...], preferred_element_type=jnp.float32)
        u = (u + bb_ref[...]) * s_ref[...] + be_ref[...]
        o_ref[...] = jnp.maximum(xb + u, 0.0)

    row = pl.BlockSpec((BR, D), lambda i: (i, 0))
    half = pl.BlockSpec((NC, BR, DH), lambda i: (0, i, 0))
    full = pl.BlockSpec((D, D), lambda i: (0, 0))
    vec = pl.BlockSpec((1, D), lambda i: (0, 0))
    return pl.pallas_call(
        body,
        grid=(NBLK,),
        in_specs=[row, half, full, vec, full, vec, vec, vec],
        out_specs=row,
        out_shape=jax.ShapeDtypeStruct((N, D), jnp.float32),
    )(x, agg, Wa, ba, Wb, bb, scale, be)


def _mlp_pool_block(h, agg, Wa, ba, Wb, bb, scale, be, batch_r):
    def body(h_ref, a_ref, Wa_ref, ba_ref, Wb_ref, bb_ref,
             s_ref, be_ref, b_ref, o_ref, acc_ref):
        i = pl.program_id(0)
        hb = h_ref[...]
        hin = hb + jnp.concatenate([a_ref[0], a_ref[1]], axis=1)
        t = jnp.dot(hin, Wa_ref[...], preferred_element_type=jnp.float32)
        t = jnp.maximum(t + ba_ref[...], 0.0)
        u = jnp.dot(t, Wb_ref[...], preferred_element_type=jnp.float32)
        u = (u + bb_ref[...]) * s_ref[...] + be_ref[...]
        h2 = jnp.maximum(hb + u, 0.0)
        seg = b_ref[0, 0, :]
        onehot = (lax.broadcasted_iota(jnp.int32, (G, BR), 0)
                  == seg[None, :]).astype(jnp.float32)
        part = jnp.dot(onehot, h2, preferred_element_type=jnp.float32)

        @pl.when(i == 0)
        def _():
            acc_ref[...] = part

        @pl.when(i > 0)
        def _():
            acc_ref[...] += part

        @pl.when(i == NBLK - 1)
        def _():
            p = acc_ref[...]
            m = jnp.max(p, axis=1, keepdims=True)
            lse = jnp.log(jnp.sum(jnp.exp(p - m), axis=1, keepdims=True)) + m
            o_ref[...] = p - lse

    row = pl.BlockSpec((BR, D), lambda i: (i, 0))
    half = pl.BlockSpec((NC, BR, DH), lambda i: (0, i, 0))
    full = pl.BlockSpec((D, D), lambda i: (0, 0))
    vec = pl.BlockSpec((1, D), lambda i: (0, 0))
    bspec = pl.BlockSpec((1, 1, BR), lambda i: (i, 0, 0))
    out = pl.BlockSpec((G, D), lambda i: (0, 0))
    return pl.pallas_call(
        body,
        grid=(NBLK,),
        in_specs=[row, half, full, vec, full, vec, vec, vec, bspec],
        out_specs=out,
        out_shape=jax.ShapeDtypeStruct((G, D), jnp.float32),
        scratch_shapes=[pltpu.VMEM((G, D), jnp.float32)],
    )(h, agg, Wa, ba, Wb, bb, scale, be, batch_r)


def kernel(x, edge_index, batch_index,
           W1a, b1a, W1b, b1b, W2a, b2a, W2b, b2b,
           g1, be1, g2, be2):
    src = edge_index[0]
    dst = edge_index[1]
    pad_e = EPAD - E
    src_r = jnp.concatenate([src, jnp.zeros((pad_e,), jnp.int32)]
                            ).reshape(NS, NCH, CL)
    dst_r = jnp.concatenate([dst, jnp.full((pad_e,), N, jnp.int32)]
                            ).reshape(NS, NCH, CL)
    zeros_hbm = jnp.zeros((NPAD, DH), jnp.float32)
    batch_r = batch_index.reshape(NBLK, 1, BR)

    s1 = (g1 * INV_BN).reshape(1, D)
    s2 = (g2 * INV_BN).reshape(1, D)

    agg1 = _sc_aggregate(x, src_r, dst_r, zeros_hbm)
    h1 = _mlp_res_block(x, agg1,
                        W1a, b1a.reshape(1, D), W1b, b1b.reshape(1, D),
                        s1, be1.reshape(1, D))
    agg2 = _sc_aggregate(h1, src_r, dst_r, zeros_hbm)
    return _mlp_pool_block(h1, agg2,
                           W2a, b2a.reshape(1, D), W2b, b2b.reshape(1, D),
                           s2, be2.reshape(1, D), batch_r)


# speedup vs baseline: 1.1280x; 1.1280x over previous
import functools

import jax
import jax.numpy as jnp
from jax import lax
from jax.experimental import pallas as pl
from jax.experimental.pallas import tpu as pltpu
from jax.experimental.pallas import tpu_sc as plsc

N = 10000
E = 320000
D = 128
G = 64
BN_EPS = 1e-5

NC = 2
NS = 16
DH = D // NC
CL = 128
NCH = 160
EPW = CL * NCH
EPAD = EPW * NS
NPAD = 10112
RPT = NPAD // NS
NLAST = N - (NS - 1) * RPT

KBUF = 4
MG = 2
NQ = 4
NCHQ = NCH // NQ

BR = 400
NBLK = N // BR
INV_BN = 1.0 / (1.0 + BN_EPS) ** 0.5


def _sc_aggregate(xn, src_r, dst_r, zeros_hbm):
    mesh = plsc.VectorSubcoreMesh(core_axis_name="c", subcore_axis_name="s")

    @functools.partial(
        pl.kernel,
        mesh=mesh,
        compiler_params=pltpu.CompilerParams(use_tc_tiling_on_sc=False),
        out_type=jax.ShapeDtypeStruct((NC, NPAD, DH), jnp.float32),
        scratch_types=[
            pltpu.VMEM((NCHQ, CL), jnp.int32),
            pltpu.VMEM((NCHQ, CL), jnp.int32),
            [pltpu.VMEM((CL, DH), jnp.float32) for _ in range(KBUF)],
            pltpu.VMEM_SHARED((NPAD, DH), jnp.float32),
            pltpu.VMEM_SHARED((NPAD, DH), jnp.float32),
            [pltpu.SemaphoreType.DMA for _ in range(KBUF)],
            [pltpu.SemaphoreType.DMA for _ in range(KBUF)],
        ],
    )
    def agg_kernel(x_hbm, src_hbm, dst_hbm, z_hbm, out_hbm,
                   src_v, dst_v, rows, tab_sh, agg_sh, gsem, ssem):
        cid = lax.axis_index("c")
        sid = lax.axis_index("s")
        rs = pl.ds(sid * RPT, RPT)
        cs = pl.ds(cid * DH, DH)
        @pl.when(sid < NS - 1)
        def _():
            pltpu.sync_copy(x_hbm.at[pl.ds(sid * RPT, RPT), cs],
                            tab_sh.at[rs])

        @pl.when(sid == NS - 1)
        def _():
            pltpu.sync_copy(x_hbm.at[pl.ds((NS - 1) * RPT, NLAST), cs],
                            tab_sh.at[pl.ds((NS - 1) * RPT, NLAST)])

        pltpu.sync_copy(z_hbm.at[rs], agg_sh.at[rs])
        plsc.subcore_barrier()

        for q in range(NQ):
            pltpu.sync_copy(src_hbm.at[sid, pl.ds(q * NCHQ, NCHQ)], src_v)
            pltpu.sync_copy(dst_hbm.at[sid, pl.ds(q * NCHQ, NCHQ)], dst_v)
            for b in range(KBUF):
                pltpu.async_copy(tab_sh.at[src_v.at[b]], rows[b], gsem[b])

            def outer(j0, carry):
                for b in range(KBUF):
                    j = j0 * KBUF + b
                    pltpu.make_async_copy(tab_sh.at[src_v.at[0]],
                                          rows[b], gsem[b]).wait()
                    pltpu.async_copy(rows[b], agg_sh.at[dst_v.at[j]],
                                     ssem[b], add=True)
                    jf = j + MG
                    bf = (b + MG) % KBUF

                    @pl.when(jnp.logical_and(jf >= KBUF, jf < NCHQ))
                    def _():
                        pltpu.make_async_copy(rows[bf],
                                              agg_sh.at[dst_v.at[0]],
                                              ssem[bf]).wait()
                        pltpu.async_copy(tab_sh.at[src_v.at[jf]],
                                         rows[bf], gsem[bf])
                return carry

            lax.fori_loop(0, NCHQ // KBUF, outer, 0)
            for b in range(KBUF):
                pltpu.make_async_copy(rows[b], agg_sh.at[dst_v.at[0]],
                                      ssem[b]).wait()
        plsc.subcore_barrier()
        pltpu.sync_copy(agg_sh.at[rs], out_hbm.at[cid, rs])

    return agg_kernel(xn, src_r, dst_r, zeros_hbm)


def _mlp_res_block(x, agg, Wa, ba, Wb, bb, scale, be):
    def body(x_ref, a_ref, Wa_ref, ba_ref, Wb_ref, bb_ref,
             s_ref, be_ref, o_ref):
        xb = x_ref[...]
        h = xb + jnp.concatenate([a_ref[0], a_ref[1]], axis=1)
        t = jnp.dot(h, Wa_ref[...], preferred_element_type=jnp.float32)
        t = jnp.maximum(t + ba_ref[...], 0.0)
        u = jnp.dot(t, Wb_ref[...], preferred_element_type=jnp.float32)
        u = (u + bb_ref[...]) * s_ref[...] + be_ref[...]
        o_ref[...] = jnp.maximum(xb + u, 0.0)

    row = pl.BlockSpec((BR, D), lambda i: (i, 0))
    half = pl.BlockSpec((NC, BR, DH), lambda i: (0, i, 0))
    full = pl.BlockSpec((D, D), lambda i: (0, 0))
    vec = pl.BlockSpec((1, D), lambda i: (0, 0))
    return pl.pallas_call(
        body,
        grid=(NBLK,),
        in_specs=[row, half, full, vec, full, vec, vec, vec],
        out_specs=row,
        out_shape=jax.ShapeDtypeStruct((N, D), jnp.float32),
    )(x, agg, Wa, ba, Wb, bb, scale, be)


def _mlp_pool_block(h, agg, Wa, ba, Wb, bb, scale, be, batch_r):
    def body(h_ref, a_ref, Wa_ref, ba_ref, Wb_ref, bb_ref,
             s_ref, be_ref, b_ref, o_ref, acc_ref):
        i = pl.program_id(0)
        hb = h_ref[...]
        hin = hb + jnp.concatenate([a_ref[0], a_ref[1]], axis=1)
        t = jnp.dot(hin, Wa_ref[...], preferred_element_type=jnp.float32)
        t = jnp.maximum(t + ba_ref[...], 0.0)
        u = jnp.dot(t, Wb_ref[...], preferred_element_type=jnp.float32)
        u = (u + bb_ref[...]) * s_ref[...] + be_ref[...]
        h2 = jnp.maximum(hb + u, 0.0)
        seg = b_ref[0, 0, :]
        onehot = (lax.broadcasted_iota(jnp.int32, (G, BR), 0)
                  == seg[None, :]).astype(jnp.float32)
        part = jnp.dot(onehot, h2, preferred_element_type=jnp.float32)

        @pl.when(i == 0)
        def _():
            acc_ref[...] = part

        @pl.when(i > 0)
        def _():
            acc_ref[...] += part

        @pl.when(i == NBLK - 1)
        def _():
            p = acc_ref[...]
            m = jnp.max(p, axis=1, keepdims=True)
            lse = jnp.log(jnp.sum(jnp.exp(p - m), axis=1, keepdims=True)) + m
            o_ref[...] = p - lse

    row = pl.BlockSpec((BR, D), lambda i: (i, 0))
    half = pl.BlockSpec((NC, BR, DH), lambda i: (0, i, 0))
    full = pl.BlockSpec((D, D), lambda i: (0, 0))
    vec = pl.BlockSpec((1, D), lambda i: (0, 0))
    bspec = pl.BlockSpec((1, 1, BR), lambda i: (i, 0, 0))
    out = pl.BlockSpec((G, D), lambda i: (0, 0))
    return pl.pallas_call(
        body,
        grid=(NBLK,),
        in_specs=[row, half, full, vec, full, vec, vec, vec, bspec],
        out_specs=out,
        out_shape=jax.ShapeDtypeStruct((G, D), jnp.float32),
        scratch_shapes=[pltpu.VMEM((G, D), jnp.float32)],
    )(h, agg, Wa, ba, Wb, bb, scale, be, batch_r)


def kernel(x, edge_index, batch_index,
           W1a, b1a, W1b, b1b, W2a, b2a, W2b, b2b,
           g1, be1, g2, be2):
    src = edge_index[0]
    dst = edge_index[1]
    pad_e = EPAD - E
    src_r = jnp.concatenate([src, jnp.zeros((pad_e,), jnp.int32)]
                            ).reshape(NS, NCH, CL)
    dst_r = jnp.concatenate([dst, jnp.full((pad_e,), N, jnp.int32)]
                            ).reshape(NS, NCH, CL)
    zeros_hbm = jnp.zeros((NPAD, DH), jnp.float32)
    batch_r = batch_index.reshape(NBLK, 1, BR)

    s1 = (g1 * INV_BN).reshape(1, D)
    s2 = (g2 * INV_BN).reshape(1, D)

    agg1 = _sc_aggregate(x, src_r, dst_r, zeros_hbm)
    h1 = _mlp_res_block(x, agg1,
                        W1a, b1a.reshape(1, D), W1b, b1b.reshape(1, D),
                        s1, be1.reshape(1, D))
    agg2 = _sc_aggregate(h1, src_r, dst_r, zeros_hbm)
    return _mlp_pool_block(h1, agg2,
                           W2a, b2a.reshape(1, D), W2b, b2b.reshape(1, D),
                           s2, be2.reshape(1, D), batch_r)


# speedup vs baseline: 1.2167x; 1.0787x over previous
import functools

import jax
import jax.numpy as jnp
from jax import lax
from jax.experimental import pallas as pl
from jax.experimental.pallas import tpu as pltpu
from jax.experimental.pallas import tpu_sc as plsc

N = 10000
E = 320000
D = 128
G = 64
BN_EPS = 1e-5

NC = 2
NS = 16
DH = D // NC
CL = 128
NCH = 160
EPW = CL * NCH
EPAD = EPW * NS
NPAD = 10112
RPT = NPAD // NS
NLAST = N - (NS - 1) * RPT

KBUF = 4
MG = 2
NQ = 4
NCHQ = NCH // NQ

BR = 2000
NBLK = N // BR
INV_BN = 1.0 / (1.0 + BN_EPS) ** 0.5


def _sc_aggregate(xn, src_r, dst_r, zeros_hbm):
    mesh = plsc.VectorSubcoreMesh(core_axis_name="c", subcore_axis_name="s")

    @functools.partial(
        pl.kernel,
        mesh=mesh,
        compiler_params=pltpu.CompilerParams(use_tc_tiling_on_sc=False),
        out_type=jax.ShapeDtypeStruct((NC, NPAD, DH), jnp.float32),
        scratch_types=[
            pltpu.VMEM((NCHQ, CL), jnp.int32),
            pltpu.VMEM((NCHQ, CL), jnp.int32),
            [pltpu.VMEM((CL, DH), jnp.float32) for _ in range(KBUF)],
            pltpu.VMEM_SHARED((NPAD, DH), jnp.float32),
            pltpu.VMEM_SHARED((NPAD, DH), jnp.float32),
            [pltpu.SemaphoreType.DMA for _ in range(KBUF)],
            [pltpu.SemaphoreType.DMA for _ in range(KBUF)],
        ],
    )
    def agg_kernel(x_hbm, src_hbm, dst_hbm, z_hbm, out_hbm,
                   src_v, dst_v, rows, tab_sh, agg_sh, gsem, ssem):
        cid = lax.axis_index("c")
        sid = lax.axis_index("s")
        rs = pl.ds(sid * RPT, RPT)
        cs = pl.ds(cid * DH, DH)
        @pl.when(sid < NS - 1)
        def _():
            pltpu.sync_copy(x_hbm.at[pl.ds(sid * RPT, RPT), cs],
                            tab_sh.at[rs])

        @pl.when(sid == NS - 1)
        def _():
            pltpu.sync_copy(x_hbm.at[pl.ds((NS - 1) * RPT, NLAST), cs],
                            tab_sh.at[pl.ds((NS - 1) * RPT, NLAST)])

        pltpu.sync_copy(z_hbm.at[rs], agg_sh.at[rs])
        plsc.subcore_barrier()

        for q in range(NQ):
            pltpu.sync_copy(src_hbm.at[sid, pl.ds(q * NCHQ, NCHQ)], src_v)
            pltpu.sync_copy(dst_hbm.at[sid, pl.ds(q * NCHQ, NCHQ)], dst_v)
            for b in range(KBUF):
                pltpu.async_copy(tab_sh.at[src_v.at[b]], rows[b], gsem[b])

            def outer(j0, carry):
                for b in range(KBUF):
                    j = j0 * KBUF + b
                    pltpu.make_async_copy(tab_sh.at[src_v.at[0]],
                                          rows[b], gsem[b]).wait()
                    pltpu.async_copy(rows[b], agg_sh.at[dst_v.at[j]],
                                     ssem[b], add=True)
                    jf = j + MG
                    bf = (b + MG) % KBUF

                    @pl.when(jnp.logical_and(jf >= KBUF, jf < NCHQ))
                    def _():
                        pltpu.make_async_copy(rows[bf],
                                              agg_sh.at[dst_v.at[0]],
                                              ssem[bf]).wait()
                        pltpu.async_copy(tab_sh.at[src_v.at[jf]],
                                         rows[bf], gsem[bf])
                return carry

            lax.fori_loop(0, NCHQ // KBUF, outer, 0)
            for b in range(KBUF):
                pltpu.make_async_copy(rows[b], agg_sh.at[dst_v.at[0]],
                                      ssem[b]).wait()
        plsc.subcore_barrier()
        pltpu.sync_copy(agg_sh.at[rs], out_hbm.at[cid, rs])

    return agg_kernel(xn, src_r, dst_r, zeros_hbm)


def _mlp_res_block(x, agg, Wa, ba, Wb, bb, scale, be):
    def body(x_ref, a_ref, Wa_ref, ba_ref, Wb_ref, bb_ref,
             s_ref, be_ref, o_ref):
        xb = x_ref[...]
        h = xb + jnp.concatenate([a_ref[0], a_ref[1]], axis=1)
        t = jnp.dot(h, Wa_ref[...], preferred_element_type=jnp.float32)
        t = jnp.maximum(t + ba_ref[...], 0.0)
        u = jnp.dot(t, Wb_ref[...], preferred_element_type=jnp.float32)
        u = (u + bb_ref[...]) * s_ref[...] + be_ref[...]
        o_ref[...] = jnp.maximum(xb + u, 0.0)

    row = pl.BlockSpec((BR, D), lambda i: (i, 0))
    half = pl.BlockSpec((NC, BR, DH), lambda i: (0, i, 0))
    full = pl.BlockSpec((D, D), lambda i: (0, 0))
    vec = pl.BlockSpec((1, D), lambda i: (0, 0))
    return pl.pallas_call(
        body,
        grid=(NBLK,),
        in_specs=[row, half, full, vec, full, vec, vec, vec],
        out_specs=row,
        out_shape=jax.ShapeDtypeStruct((N, D), jnp.float32),
    )(x, agg, Wa, ba, Wb, bb, scale, be)


def _mlp_pool_block(h, agg, Wa, ba, Wb, bb, scale, be, batch_r):
    def body(h_ref, a_ref, Wa_ref, ba_ref, Wb_ref, bb_ref,
             s_ref, be_ref, b_ref, o_ref, acc_ref):
        i = pl.program_id(0)
        hb = h_ref[...]
        hin = hb + jnp.concatenate([a_ref[0], a_ref[1]], axis=1)
        t = jnp.dot(hin, Wa_ref[...], preferred_element_type=jnp.float32)
        t = jnp.maximum(t + ba_ref[...], 0.0)
        u = jnp.dot(t, Wb_ref[...], preferred_element_type=jnp.float32)
        u = (u + bb_ref[...]) * s_ref[...] + be_ref[...]
        h2 = jnp.maximum(hb + u, 0.0)
        seg = b_ref[0, 0, :]
        onehot = (lax.broadcasted_iota(jnp.int32, (G, BR), 0)
                  == seg[None, :]).astype(jnp.float32)
        part = jnp.dot(onehot, h2, preferred_element_type=jnp.float32)

        @pl.when(i == 0)
        def _():
            acc_ref[...] = part

        @pl.when(i > 0)
        def _():
            acc_ref[...] += part

        @pl.when(i == NBLK - 1)
        def _():
            p = acc_ref[...]
            m = jnp.max(p, axis=1, keepdims=True)
            lse = jnp.log(jnp.sum(jnp.exp(p - m), axis=1, keepdims=True)) + m
            o_ref[...] = p - lse

    row = pl.BlockSpec((BR, D), lambda i: (i, 0))
    half = pl.BlockSpec((NC, BR, DH), lambda i: (0, i, 0))
    full = pl.BlockSpec((D, D), lambda i: (0, 0))
    vec = pl.BlockSpec((1, D), lambda i: (0, 0))
    bspec = pl.BlockSpec((1, 1, BR), lambda i: (i, 0, 0))
    out = pl.BlockSpec((G, D), lambda i: (0, 0))
    return pl.pallas_call(
        body,
        grid=(NBLK,),
        in_specs=[row, half, full, vec, full, vec, vec, vec, bspec],
        out_specs=out,
        out_shape=jax.ShapeDtypeStruct((G, D), jnp.float32),
        scratch_shapes=[pltpu.VMEM((G, D), jnp.float32)],
    )(h, agg, Wa, ba, Wb, bb, scale, be, batch_r)


def kernel(x, edge_index, batch_index,
           W1a, b1a, W1b, b1b, W2a, b2a, W2b, b2b,
           g1, be1, g2, be2):
    src = edge_index[0]
    dst = edge_index[1]
    pad_e = EPAD - E
    src_r = jnp.concatenate([src, jnp.zeros((pad_e,), jnp.int32)]
                            ).reshape(NS, NCH, CL)
    dst_r = jnp.concatenate([dst, jnp.full((pad_e,), N, jnp.int32)]
                            ).reshape(NS, NCH, CL)
    zeros_hbm = jnp.zeros((NPAD, DH), jnp.float32)
    batch_r = batch_index.reshape(NBLK, 1, BR)

    s1 = (g1 * INV_BN).reshape(1, D)
    s2 = (g2 * INV_BN).reshape(1, D)

    agg1 = _sc_aggregate(x, src_r, dst_r, zeros_hbm)
    h1 = _mlp_res_block(x, agg1,
                        W1a, b1a.reshape(1, D), W1b, b1b.reshape(1, D),
                        s1, be1.reshape(1, D))
    agg2 = _sc_aggregate(h1, src_r, dst_r, zeros_hbm)
    return _mlp_pool_block(h1, agg2,
                           W2a, b2a.reshape(1, D), W2b, b2b.reshape(1, D),
                           s2, be2.reshape(1, D), batch_r)


# speedup vs baseline: 1.2226x; 1.0048x over previous
import functools

import jax
import jax.numpy as jnp
from jax import lax
from jax.experimental import pallas as pl
from jax.experimental.pallas import tpu as pltpu
from jax.experimental.pallas import tpu_sc as plsc

N = 10000
E = 320000
D = 128
G = 64
BN_EPS = 1e-5

NC = 2
NS = 16
DH = D // NC
CL = 128
NCH = 160
EPW = CL * NCH
EPAD = EPW * NS
NPAD = 10112
RPT = NPAD // NS
NLAST = N - (NS - 1) * RPT

KBUF = 4
MG = 2
NQ = 4
NCHQ = NCH // NQ

BR = 5000
NBLK = N // BR
INV_BN = 1.0 / (1.0 + BN_EPS) ** 0.5


def _sc_aggregate(xn, src_r, dst_r, zeros_hbm):
    mesh = plsc.VectorSubcoreMesh(core_axis_name="c", subcore_axis_name="s")

    @functools.partial(
        pl.kernel,
        mesh=mesh,
        compiler_params=pltpu.CompilerParams(use_tc_tiling_on_sc=False),
        out_type=jax.ShapeDtypeStruct((NC, NPAD, DH), jnp.float32),
        scratch_types=[
            pltpu.VMEM((NCHQ, CL), jnp.int32),
            pltpu.VMEM((NCHQ, CL), jnp.int32),
            [pltpu.VMEM((CL, DH), jnp.float32) for _ in range(KBUF)],
            pltpu.VMEM_SHARED((NPAD, DH), jnp.float32),
            pltpu.VMEM_SHARED((NPAD, DH), jnp.float32),
            [pltpu.SemaphoreType.DMA for _ in range(KBUF)],
            [pltpu.SemaphoreType.DMA for _ in range(KBUF)],
        ],
    )
    def agg_kernel(x_hbm, src_hbm, dst_hbm, z_hbm, out_hbm,
                   src_v, dst_v, rows, tab_sh, agg_sh, gsem, ssem):
        cid = lax.axis_index("c")
        sid = lax.axis_index("s")
        rs = pl.ds(sid * RPT, RPT)
        cs = pl.ds(cid * DH, DH)
        @pl.when(sid < NS - 1)
        def _():
            pltpu.sync_copy(x_hbm.at[pl.ds(sid * RPT, RPT), cs],
                            tab_sh.at[rs])

        @pl.when(sid == NS - 1)
        def _():
            pltpu.sync_copy(x_hbm.at[pl.ds((NS - 1) * RPT, NLAST), cs],
                            tab_sh.at[pl.ds((NS - 1) * RPT, NLAST)])

        pltpu.sync_copy(z_hbm.at[rs], agg_sh.at[rs])
        plsc.subcore_barrier()

        for q in range(NQ):
            pltpu.sync_copy(src_hbm.at[sid, pl.ds(q * NCHQ, NCHQ)], src_v)
            pltpu.sync_copy(dst_hbm.at[sid, pl.ds(q * NCHQ, NCHQ)], dst_v)
            for b in range(KBUF):
                pltpu.async_copy(tab_sh.at[src_v.at[b]], rows[b], gsem[b])

            def outer(j0, carry):
                for b in range(KBUF):
                    j = j0 * KBUF + b
                    pltpu.make_async_copy(tab_sh.at[src_v.at[0]],
                                          rows[b], gsem[b]).wait()
                    pltpu.async_copy(rows[b], agg_sh.at[dst_v.at[j]],
                                     ssem[b], add=True)
                    jf = j + MG
                    bf = (b + MG) % KBUF

                    @pl.when(jnp.logical_and(jf >= KBUF, jf < NCHQ))
                    def _():
                        pltpu.make_async_copy(rows[bf],
                                              agg_sh.at[dst_v.at[0]],
                                              ssem[bf]).wait()
                        pltpu.async_copy(tab_sh.at[src_v.at[jf]],
                                         rows[bf], gsem[bf])
                return carry

            lax.fori_loop(0, NCHQ // KBUF, outer, 0)
            for b in range(KBUF):
                pltpu.make_async_copy(rows[b], agg_sh.at[dst_v.at[0]],
                                      ssem[b]).wait()
        plsc.subcore_barrier()
        pltpu.sync_copy(agg_sh.at[rs], out_hbm.at[cid, rs])

    return agg_kernel(xn, src_r, dst_r, zeros_hbm)


def _mlp_res_block(x, agg, Wa, ba, Wb, bb, scale, be):
    def body(x_ref, a_ref, Wa_ref, ba_ref, Wb_ref, bb_ref,
             s_ref, be_ref, o_ref):
        xb = x_ref[...]
        h = xb + jnp.concatenate([a_ref[0], a_ref[1]], axis=1)
        t = jnp.dot(h, Wa_ref[...], preferred_element_type=jnp.float32)
        t = jnp.maximum(t + ba_ref[...], 0.0)
        u = jnp.dot(t, Wb_ref[...], preferred_element_type=jnp.float32)
        u = (u + bb_ref[...]) * s_ref[...] + be_ref[...]
        o_ref[...] = jnp.maximum(xb + u, 0.0)

    row = pl.BlockSpec((BR, D), lambda i: (i, 0))
    half = pl.BlockSpec((NC, BR, DH), lambda i: (0, i, 0))
    full = pl.BlockSpec((D, D), lambda i: (0, 0))
    vec = pl.BlockSpec((1, D), lambda i: (0, 0))
    return pl.pallas_call(
        body,
        grid=(NBLK,),
        in_specs=[row, half, full, vec, full, vec, vec, vec],
        out_specs=row,
        out_shape=jax.ShapeDtypeStruct((N, D), jnp.float32),
    )(x, agg, Wa, ba, Wb, bb, scale, be)


def _mlp_pool_block(h, agg, Wa, ba, Wb, bb, scale, be, batch_r):
    def body(h_ref, a_ref, Wa_ref, ba_ref, Wb_ref, bb_ref,
             s_ref, be_ref, b_ref, o_ref, acc_ref):
        i = pl.program_id(0)
        hb = h_ref[...]
        hin = hb + jnp.concatenate([a_ref[0], a_ref[1]], axis=1)
        t = jnp.dot(hin, Wa_ref[...], preferred_element_type=jnp.float32)
        t = jnp.maximum(t + ba_ref[...], 0.0)
        u = jnp.dot(t, Wb_ref[...], preferred_element_type=jnp.float32)
        u = (u + bb_ref[...]) * s_ref[...] + be_ref[...]
        h2 = jnp.maximum(hb + u, 0.0)
        seg = b_ref[0, 0, :]
        onehot = (lax.broadcasted_iota(jnp.int32, (G, BR), 0)
                  == seg[None, :]).astype(jnp.float32)
        part = jnp.dot(onehot, h2, preferred_element_type=jnp.float32)

        @pl.when(i == 0)
        def _():
            acc_ref[...] = part

        @pl.when(i > 0)
        def _():
            acc_ref[...] += part

        @pl.when(i == NBLK - 1)
        def _():
            p = acc_ref[...]
            m = jnp.max(p, axis=1, keepdims=True)
            lse = jnp.log(jnp.sum(jnp.exp(p - m), axis=1, keepdims=True)) + m
            o_ref[...] = p - lse

    row = pl.BlockSpec((BR, D), lambda i: (i, 0))
    half = pl.BlockSpec((NC, BR, DH), lambda i: (0, i, 0))
    full = pl.BlockSpec((D, D), lambda i: (0, 0))
    vec = pl.BlockSpec((1, D), lambda i: (0, 0))
    bspec = pl.BlockSpec((1, 1, BR), lambda i: (i, 0, 0))
    out = pl.BlockSpec((G, D), lambda i: (0, 0))
    return pl.pallas_call(
        body,
        grid=(NBLK,),
        in_specs=[row, half, full, vec, full, vec, vec, vec, bspec],
        out_specs=out,
        out_shape=jax.ShapeDtypeStruct((G, D), jnp.float32),
        scratch_shapes=[pltpu.VMEM((G, D), jnp.float32)],
    )(h, agg, Wa, ba, Wb, bb, scale, be, batch_r)


def kernel(x, edge_index, batch_index,
           W1a, b1a, W1b, b1b, W2a, b2a, W2b, b2b,
           g1, be1, g2, be2):
    src = edge_index[0]
    dst = edge_index[1]
    pad_e = EPAD - E
    src_r = jnp.concatenate([src, jnp.zeros((pad_e,), jnp.int32)]
                            ).reshape(NS, NCH, CL)
    dst_r = jnp.concatenate([dst, jnp.full((pad_e,), N, jnp.int32)]
                            ).reshape(NS, NCH, CL)
    zeros_hbm = jnp.zeros((NPAD, DH), jnp.float32)
    batch_r = batch_index.reshape(NBLK, 1, BR)

    s1 = (g1 * INV_BN).reshape(1, D)
    s2 = (g2 * INV_BN).reshape(1, D)

    agg1 = _sc_aggregate(x, src_r, dst_r, zeros_hbm)
    h1 = _mlp_res_block(x, agg1,
                        W1a, b1a.reshape(1, D), W1b, b1b.reshape(1, D),
                        s1, be1.reshape(1, D))
    agg2 = _sc_aggregate(h1, src_r, dst_r, zeros_hbm)
    return _mlp_pool_block(h1, agg2,
                           W2a, b2a.reshape(1, D), W2b, b2b.reshape(1, D),
                           s2, be2.reshape(1, D), batch_r)
